# baseline (speedup 1.0000x reference)
import functools
import math

import jax
import jax.numpy as jnp
from jax import lax
from jax.experimental import pallas as pl
from jax.experimental.pallas import tpu as pltpu
from jax.experimental.pallas import tpu_sc as plsc

VOCAB = 100000
D = 768
NUM_LANG = 4
B = 4
S = 4096
N = B * S
SCALE = math.sqrt(D)

NC = 2
NS = 16
NW = NC * NS
S_PER_W = S // NW
PE_CHUNK = 32
N_SCHUNK = S_PER_W // PE_CHUNK
CHUNK = 32
SUB = PE_CHUNK // CHUNK
NT = N_SCHUNK * B * SUB
DV = D // 16


def _pos_table():
    pos = jnp.arange(0, S, dtype=jnp.float32)[:, None]
    div = jnp.exp(jnp.arange(0, D, 2, dtype=jnp.float32) * (-math.log(10000.0) / D))
    pe = jnp.zeros((S, D), dtype=jnp.float32)
    pe = pe.at[:, 0::2].set(jnp.sin(pos * div))
    pe = pe.at[:, 1::2].set(jnp.cos(pos * div))
    return pe


def _proj_body(wl_ref, wp_ref, o_ref):
    o_ref[...] = lax.dot_general(
        wl_ref[...], wp_ref[...], (((1,), (1,)), ((), ())),
        preferred_element_type=jnp.float32)


_lang_proj = pl.pallas_call(
    _proj_body,
    out_shape=jax.ShapeDtypeStruct((NUM_LANG, D), jnp.float32),
)


def _chunk_parts(t):
    sc, rem = divmod(t, B * SUB)
    b, c2 = divmod(rem, SUB)
    return sc, b, c2


@functools.partial(
    pl.kernel,
    out_type=jax.ShapeDtypeStruct((N, D), jnp.float32),
    mesh=plsc.VectorSubcoreMesh(core_axis_name="c", subcore_axis_name="s"),
    scratch_types=(
        [pltpu.VMEM((B * S_PER_W,), jnp.int32)] * 2
        + [pltpu.VMEM((CHUNK, D), jnp.float32)] * 4
        + [pltpu.VMEM((PE_CHUNK, D), jnp.float32)]
        + [pltpu.SemaphoreType.DMA] * 7
    ),
)
def _sc_embed(wtok_hbm, tokid_hbm, langid_hbm, ltab_hbm, pe_hbm, out_hbm,
              idx_v, lidx_v, rows0, rows1, langb0, langb1, pe_v,
              gs0, gs1, ls0, ls1, ss0, ss1, pes):
    rows = (rows0, rows1)
    langb = (langb0, langb1)
    gsem = (gs0, gs1)
    lsem = (ls0, ls1)
    ssem = (ss0, ss1)
    wid = lax.axis_index("s") * NC + lax.axis_index("c")
    s_lo = wid * S_PER_W
    for b in range(B):
        pltpu.sync_copy(tokid_hbm.at[pl.ds(b * S + s_lo, S_PER_W)],
                        idx_v.at[pl.ds(b * S_PER_W, S_PER_W)])
        pltpu.sync_copy(langid_hbm.at[pl.ds(b * S + s_lo, S_PER_W)],
                        lidx_v.at[pl.ds(b * S_PER_W, S_PER_W)])

    def idx_off(t):
        sc, b, c2 = _chunk_parts(t)
        return b * S_PER_W + sc * PE_CHUNK + c2 * CHUNK

    def start_gathers(t):
        k = t % 2
        off = idx_off(t)
        g = pltpu.async_copy(
            wtok_hbm.at[idx_v.at[pl.ds(off, CHUNK)]], rows[k], gsem[k])
        l = pltpu.async_copy(
            ltab_hbm.at[lidx_v.at[pl.ds(off, CHUNK)]], langb[k], lsem[k])
        return g, l

    g_cp = [None] * NT
    st_cp = [None] * NT
    g_cp[0] = start_gathers(0)
    pe_cp = pltpu.async_copy(pe_hbm.at[pl.ds(s_lo, PE_CHUNK)], pe_v, pes)
    for t in range(NT):
        k = t % 2
        sc, b, c2 = _chunk_parts(t)
        if t + 1 < NT:
            if t >= 1:
                st_cp[t - 1].wait()
            g_cp[t + 1] = start_gathers(t + 1)
        if t % (B * SUB) == 0:
            pe_cp.wait()
        g_cp[t][0].wait()
        g_cp[t][1].wait()
        rk, lk = rows[k], langb[k]

        def row_body(i, _, rk=rk, lk=lk, pe_off=c2 * CHUNK):
            def j_body(jb, _2):
                for j in range(8):
                    sl = pl.ds(jb * 128 + j * 16, 16)
                    rk[i, sl] = (rk[i, sl] * SCALE + lk[i, sl]
                                 + pe_v[pe_off + i, sl])
                return 0

            lax.fori_loop(0, DV // 8, j_body, 0)
            return 0

        lax.fori_loop(0, CHUNK, row_body, 0)
        out_row = b * S + s_lo + sc * PE_CHUNK + c2 * CHUNK
        st_cp[t] = pltpu.async_copy(rk, out_hbm.at[pl.ds(out_row, CHUNK)],
                                    ssem[k])
        if t % (B * SUB) == B * SUB - 1 and t + 1 < NT:
            sc_next = (t + 1) // (B * SUB)
            pe_cp = pltpu.async_copy(
                pe_hbm.at[pl.ds(s_lo + sc_next * PE_CHUNK, PE_CHUNK)],
                pe_v, pes)
    st_cp[NT - 2].wait()
    st_cp[NT - 1].wait()


def kernel(token_ids, lang_ids, W_tok, W_lang, W_proj):
    tok_flat = token_ids.reshape(-1).astype(jnp.int32)
    lang_flat = lang_ids.reshape(-1).astype(jnp.int32)
    ltab = _lang_proj(W_lang, W_proj)
    pe = _pos_table()
    out = _sc_embed(W_tok, tok_flat, lang_flat, ltab, pe)
    return out.reshape(B, S, D)


# speedup vs baseline: 1.0369x; 1.0369x over previous
import functools
import math

import jax
import jax.numpy as jnp
from jax import lax
from jax.experimental import pallas as pl
from jax.experimental.pallas import tpu as pltpu
from jax.experimental.pallas import tpu_sc as plsc

VOCAB = 100000
D = 768
NUM_LANG = 4
B = 4
S = 4096
N = B * S
SCALE = math.sqrt(D)

NC = 2
NS = 16
NW = NC * NS
S_PER_W = S // NW
PE_CHUNK = 32
N_SCHUNK = S_PER_W // PE_CHUNK
CHUNK = 32
SUB = PE_CHUNK // CHUNK
NT = N_SCHUNK * B * SUB
DV = D // 16


def _pos_table():
    pos = jnp.arange(0, S, dtype=jnp.float32)[:, None]
    div = jnp.exp(jnp.arange(0, D, 2, dtype=jnp.float32) * (-math.log(10000.0) / D))
    pe = jnp.zeros((S, D), dtype=jnp.float32)
    pe = pe.at[:, 0::2].set(jnp.sin(pos * div))
    pe = pe.at[:, 1::2].set(jnp.cos(pos * div))
    return pe


def _proj_body(wl_ref, wp_ref, o_ref):
    o_ref[...] = lax.dot_general(
        wl_ref[...], wp_ref[...], (((1,), (1,)), ((), ())),
        preferred_element_type=jnp.float32)


_lang_proj = pl.pallas_call(
    _proj_body,
    out_shape=jax.ShapeDtypeStruct((NUM_LANG, D), jnp.float32),
)


def _chunk_parts(t):
    sc, rem = divmod(t, B * SUB)
    b, c2 = divmod(rem, SUB)
    return sc, b, c2


@functools.partial(
    pl.kernel,
    out_type=jax.ShapeDtypeStruct((N, D), jnp.float32),
    mesh=plsc.VectorSubcoreMesh(core_axis_name="c", subcore_axis_name="s"),
    scratch_types=(
        [pltpu.VMEM((B * S_PER_W,), jnp.int32)] * 2
        + [pltpu.VMEM((CHUNK, D), jnp.float32)] * 4
        + [pltpu.VMEM((PE_CHUNK, D), jnp.float32)]
        + [pltpu.SemaphoreType.DMA] * 7
    ),
)
def _sc_embed(wtok_hbm, tokid_hbm, langid_hbm, ltab_hbm, pe_hbm, out_hbm,
              idx_v, lidx_v, rows0, rows1, langb0, langb1, pe_v,
              gs0, gs1, ls0, ls1, ss0, ss1, pes):
    rows = (rows0, rows1)
    langb = (langb0, langb1)
    gsem = (gs0, gs1)
    lsem = (ls0, ls1)
    ssem = (ss0, ss1)
    wid = lax.axis_index("s") * NC + lax.axis_index("c")
    s_lo = wid * S_PER_W
    for b in range(B):
        pltpu.sync_copy(tokid_hbm.at[pl.ds(b * S + s_lo, S_PER_W)],
                        idx_v.at[pl.ds(b * S_PER_W, S_PER_W)])
        pltpu.sync_copy(langid_hbm.at[pl.ds(b * S + s_lo, S_PER_W)],
                        lidx_v.at[pl.ds(b * S_PER_W, S_PER_W)])

    def idx_off(t):
        sc, b, c2 = _chunk_parts(t)
        return b * S_PER_W + sc * PE_CHUNK + c2 * CHUNK

    def start_gathers(t):
        k = t % 2
        off = idx_off(t)
        g = pltpu.async_copy(
            wtok_hbm.at[idx_v.at[pl.ds(off, CHUNK)]], rows[k], gsem[k])
        l = pltpu.async_copy(
            ltab_hbm.at[lidx_v.at[pl.ds(off, CHUNK)]], langb[k], lsem[k])
        return g, l

    g_cp = [None] * NT
    st_cp = [None] * NT
    g_cp[0] = start_gathers(0)
    pe_cp = pltpu.async_copy(pe_hbm.at[pl.ds(s_lo, PE_CHUNK)], pe_v, pes)
    for t in range(NT):
        k = t % 2
        sc, b, c2 = _chunk_parts(t)
        if t + 1 < NT:
            if t >= 1:
                st_cp[t - 1].wait()
            g_cp[t + 1] = start_gathers(t + 1)
        if t % (B * SUB) == 0:
            pe_cp.wait()
        g_cp[t][0].wait()
        g_cp[t][1].wait()
        rk, lk = rows[k], langb[k]

        @plsc.parallel_loop(0, CHUNK)
        def row_body(i, rk=rk, lk=lk, pe_off=c2 * CHUNK):
            @plsc.parallel_loop(0, DV // 4, unroll=2)
            def j_body(jb):
                for j in range(4):
                    sl = pl.ds(jb * 64 + j * 16, 16)
                    rk[i, sl] = (rk[i, sl] * SCALE + lk[i, sl]
                                 + pe_v[pe_off + i, sl])
        out_row = b * S + s_lo + sc * PE_CHUNK + c2 * CHUNK
        st_cp[t] = pltpu.async_copy(rk, out_hbm.at[pl.ds(out_row, CHUNK)],
                                    ssem[k])
        if t % (B * SUB) == B * SUB - 1 and t + 1 < NT:
            sc_next = (t + 1) // (B * SUB)
            pe_cp = pltpu.async_copy(
                pe_hbm.at[pl.ds(s_lo + sc_next * PE_CHUNK, PE_CHUNK)],
                pe_v, pes)
    st_cp[NT - 2].wait()
    st_cp[NT - 1].wait()


def kernel(token_ids, lang_ids, W_tok, W_lang, W_proj):
    tok_flat = token_ids.reshape(-1).astype(jnp.int32)
    lang_flat = lang_ids.reshape(-1).astype(jnp.int32)
    ltab = _lang_proj(W_lang, W_proj)
    pe = _pos_table()
    out = _sc_embed(W_tok, tok_flat, lang_flat, ltab, pe)
    return out.reshape(B, S, D)


# speedup vs baseline: 1.9613x; 1.8915x over previous
import functools
import math

import jax
import jax.numpy as jnp
from jax import lax
from jax.experimental import pallas as pl
from jax.experimental.pallas import tpu as pltpu
from jax.experimental.pallas import tpu_sc as plsc

VOCAB = 100000
D = 768
NUM_LANG = 4
B = 4
S = 4096
N = B * S
SCALE = math.sqrt(D)

NC = 2
NS = 16
NW = NC * NS
S_PER_W = S // NW
CHUNK = 16
NBUF = 4
PF = 2
N_SCHUNK = S_PER_W // CHUNK
NT = N_SCHUNK * B
DV = D // 16
LREP = 128


def _pos_table():
    pos = jnp.arange(0, S, dtype=jnp.float32)[:, None]
    div = jnp.exp(jnp.arange(0, D, 2, dtype=jnp.float32) * (-math.log(10000.0) / D))
    pe = jnp.zeros((S, D), dtype=jnp.float32)
    pe = pe.at[:, 0::2].set(jnp.sin(pos * div))
    pe = pe.at[:, 1::2].set(jnp.cos(pos * div))
    return pe


def _proj_body(wl_ref, wp_ref, o_ref):
    o_ref[...] = lax.dot_general(
        wl_ref[...], wp_ref[...], (((1,), (1,)), ((), ())),
        preferred_element_type=jnp.float32)


_lang_proj = pl.pallas_call(
    _proj_body,
    out_shape=jax.ShapeDtypeStruct((NUM_LANG, D), jnp.float32),
)


@functools.partial(
    pl.kernel,
    out_type=jax.ShapeDtypeStruct((N, D), jnp.float32),
    mesh=plsc.VectorSubcoreMesh(core_axis_name="c", subcore_axis_name="s"),
    scratch_types=(
        [pltpu.VMEM((B * S_PER_W,), jnp.int32)] * 2
        + [pltpu.VMEM((CHUNK, D), jnp.float32)] * (2 * NBUF)
        + [pltpu.VMEM((CHUNK, D), jnp.float32)]
        + [pltpu.SemaphoreType.DMA] * (3 * NBUF + 1)
    ),
)
def _sc_embed(wtok_hbm, tokid_hbm, langid_hbm, ltab_hbm, pe_hbm, out_hbm,
              idx_v, lidx_v, *bufs_and_sems):
    rows = bufs_and_sems[0:NBUF]
    langb = bufs_and_sems[NBUF:2 * NBUF]
    pe_v = bufs_and_sems[2 * NBUF]
    gsem = bufs_and_sems[2 * NBUF + 1:2 * NBUF + 1 + NBUF]
    lsem = bufs_and_sems[2 * NBUF + 1 + NBUF:2 * NBUF + 1 + 2 * NBUF]
    ssem = bufs_and_sems[2 * NBUF + 1 + 2 * NBUF:2 * NBUF + 1 + 3 * NBUF]
    pes = bufs_and_sems[-1]
    wid = lax.axis_index("s") * NC + lax.axis_index("c")
    s_lo = wid * S_PER_W
    for b in range(B):
        pltpu.sync_copy(tokid_hbm.at[pl.ds(b * S + s_lo, S_PER_W)],
                        idx_v.at[pl.ds(b * S_PER_W, S_PER_W)])
        pltpu.sync_copy(langid_hbm.at[pl.ds(b * S + s_lo, S_PER_W)],
                        lidx_v.at[pl.ds(b * S_PER_W, S_PER_W)])
    col0 = lax.iota(jnp.int32, 16)
    for v in range(B * S_PER_W // 16):
        sl = pl.ds(v * 16, 16)
        spread = col0 + ((v * 16) % LREP)
        lidx_v[sl] = lidx_v[sl] * LREP + spread

    def _parts(t):
        return divmod(t, B)

    def idx_off(t):
        sc, b = _parts(t)
        return b * S_PER_W + sc * CHUNK

    def start_gathers(t):
        k = t % NBUF
        off = idx_off(t)
        g = pltpu.async_copy(
            wtok_hbm.at[idx_v.at[pl.ds(off, CHUNK)]], rows[k], gsem[k])
        l = pltpu.async_copy(
            ltab_hbm.at[lidx_v.at[pl.ds(off, CHUNK)]], langb[k], lsem[k])
        return g, l

    g_cp = [None] * NT
    st_cp = [None] * NT
    for t in range(PF):
        g_cp[t] = start_gathers(t)
    pe_cp = pltpu.async_copy(pe_hbm.at[pl.ds(s_lo, CHUNK)], pe_v, pes)
    for t in range(NT):
        k = t % NBUF
        sc, b = _parts(t)
        if t + PF < NT:
            if t >= NBUF - PF:
                st_cp[t - (NBUF - PF)].wait()
            g_cp[t + PF] = start_gathers(t + PF)
        if t % B == 0:
            pe_cp.wait()
        g_cp[t][0].wait()
        g_cp[t][1].wait()
        rk, lk = rows[k], langb[k]

        @plsc.parallel_loop(0, CHUNK)
        def row_body(i, rk=rk, lk=lk):
            @plsc.parallel_loop(0, DV // 4, unroll=2)
            def j_body(jb):
                for j in range(4):
                    sl = pl.ds(jb * 64 + j * 16, 16)
                    rk[i, sl] = (rk[i, sl] * SCALE + lk[i, sl] + pe_v[i, sl])

        out_row = b * S + s_lo + sc * CHUNK
        st_cp[t] = pltpu.async_copy(rk, out_hbm.at[pl.ds(out_row, CHUNK)],
                                    ssem[k])
        if t % B == B - 1 and t + 1 < NT:
            pe_cp = pltpu.async_copy(
                pe_hbm.at[pl.ds(s_lo + (sc + 1) * CHUNK, CHUNK)], pe_v, pes)
    for t in range(NT - NBUF, NT):
        st_cp[t].wait()


def kernel(token_ids, lang_ids, W_tok, W_lang, W_proj):
    tok_flat = token_ids.reshape(-1).astype(jnp.int32)
    lang_flat = lang_ids.reshape(-1).astype(jnp.int32)
    ltab = _lang_proj(W_lang, W_proj)
    ltab_rep = jnp.repeat(ltab, LREP, axis=0)
    pe = _pos_table()
    out = _sc_embed(W_tok, tok_flat, lang_flat, ltab_rep, pe)
    return out.reshape(B, S, D)


# speedup vs baseline: 2.3795x; 1.2132x over previous
import functools
import math

import jax
import jax.numpy as jnp
from jax import lax
from jax.experimental import pallas as pl
from jax.experimental.pallas import tpu as pltpu
from jax.experimental.pallas import tpu_sc as plsc

VOCAB = 100000
D = 768
NUM_LANG = 4
B = 4
S = 4096
N = B * S
SCALE = math.sqrt(D)

NSL = 2
S_SL = S // NSL
N_SL = B * S_SL

NC = 2
NS = 16
NW = NC * NS
R_PER_W = N_SL // NW
CHUNK = 32
NBUF = 4
PF = 2
NCHUNK = R_PER_W // CHUNK

BLK = 512
NBLK_SL = S_SL // BLK
NBLK = S // BLK


def _pos_table():
    pos = jnp.arange(0, S, dtype=jnp.float32)[:, None]
    div = jnp.exp(jnp.arange(0, D, 2, dtype=jnp.float32) * (-math.log(10000.0) / D))
    pe = jnp.zeros((S, D), dtype=jnp.float32)
    pe = pe.at[:, 0::2].set(jnp.sin(pos * div))
    pe = pe.at[:, 1::2].set(jnp.cos(pos * div))
    return pe


def _proj_body(wl_ref, wp_ref, o_ref):
    o_ref[...] = lax.dot_general(
        wl_ref[...], wp_ref[...], (((1,), (1,)), ((), ())),
        preferred_element_type=jnp.float32)


_lang_proj = pl.pallas_call(
    _proj_body,
    out_shape=jax.ShapeDtypeStruct((NUM_LANG, D), jnp.float32),
)


@functools.partial(
    pl.kernel,
    out_type=jax.ShapeDtypeStruct((N_SL, D), jnp.float32),
    mesh=plsc.VectorSubcoreMesh(core_axis_name="c", subcore_axis_name="s"),
    scratch_types=(
        [pltpu.VMEM((R_PER_W,), jnp.int32)]
        + [pltpu.VMEM((CHUNK, D), jnp.float32)] * NBUF
        + [pltpu.SemaphoreType.DMA] * (2 * NBUF)
    ),
)
def _sc_gather(wtok_hbm, tokid_hbm, g_hbm, idx_v, *bufs_and_sems):
    bufs = bufs_and_sems[:NBUF]
    gsems = bufs_and_sems[NBUF:2 * NBUF]
    ssems = bufs_and_sems[2 * NBUF:]
    wid = lax.axis_index("s") * NC + lax.axis_index("c")
    base = wid * R_PER_W
    pltpu.sync_copy(tokid_hbm.at[pl.ds(base, R_PER_W)], idx_v)
    g_cp = [None] * NCHUNK
    st_cp = [None] * NCHUNK
    for c in range(PF):
        g_cp[c] = pltpu.async_copy(
            wtok_hbm.at[idx_v.at[pl.ds(c * CHUNK, CHUNK)]], bufs[c % NBUF],
            gsems[c % NBUF])
    for c in range(NCHUNK):
        k = c % NBUF
        if c + PF < NCHUNK:
            if c >= NBUF - PF:
                st_cp[c - (NBUF - PF)].wait()
            kk = (c + PF) % NBUF
            g_cp[c + PF] = pltpu.async_copy(
                wtok_hbm.at[idx_v.at[pl.ds((c + PF) * CHUNK, CHUNK)]],
                bufs[kk], gsems[kk])
        g_cp[c].wait()
        st_cp[c] = pltpu.async_copy(
            bufs[k], g_hbm.at[pl.ds(base + c * CHUNK, CHUNK)], ssems[k])
    for c in range(max(0, NCHUNK - NBUF), NCHUNK):
        st_cp[c].wait()


def _combine_body(lid_ref, ltab_ref, g_ref, pe_ref, o_ref):
    ids_row = lid_ref[0]
    oh = (lax.broadcasted_iota(jnp.int32, (NUM_LANG, BLK), 0)
          == jnp.broadcast_to(ids_row, (NUM_LANG, BLK))).astype(jnp.float32)
    lang = lax.dot_general(oh, ltab_ref[...], (((0,), (0,)), ((), ())),
                           preferred_element_type=jnp.float32)
    o_ref[...] = (g_ref[...] * SCALE + lang
                  + pe_ref[...].astype(jnp.float32))


def _make_combine(sl, aliased):
    kw = {}
    specs = [
        pl.BlockSpec((1, 1, BLK),
                     lambda i, b: (b * NBLK + sl * NBLK_SL + i, 0, 0)),
        pl.BlockSpec((NUM_LANG, D), lambda i, b: (0, 0)),
        pl.BlockSpec((BLK, D), lambda i, b: (b * NBLK_SL + i, 0)),
        pl.BlockSpec((BLK, D), lambda i, b: (sl * NBLK_SL + i, 0)),
    ]
    out_spec = pl.BlockSpec((BLK, D),
                            lambda i, b: (b * NBLK + sl * NBLK_SL + i, 0))
    if aliased:
        specs = [pl.BlockSpec(
            (BLK, D), lambda i, b: (b * NBLK + sl * NBLK_SL + i, 0))] + specs
        kw["input_output_aliases"] = {0: 0}

        def body(prev_ref, lid_ref, ltab_ref, g_ref, pe_ref, o_ref):
            del prev_ref
            _combine_body(lid_ref, ltab_ref, g_ref, pe_ref, o_ref)
    else:
        body = _combine_body

    return pl.pallas_call(
        body,
        grid=(NBLK_SL, B),
        in_specs=specs,
        out_specs=out_spec,
        out_shape=jax.ShapeDtypeStruct((N, D), jnp.float32),
        **kw,
    )


_combine_slice = [_make_combine(sl, aliased=(sl > 0)) for sl in range(NSL)]


def kernel(token_ids, lang_ids, W_tok, W_lang, W_proj):
    lang_r = lang_ids.reshape(-1).astype(jnp.int32).reshape(B * NBLK, 1, BLK)
    ltab = _lang_proj(W_lang, W_proj)
    pe16 = _pos_table().astype(jnp.bfloat16)
    tok3 = token_ids.astype(jnp.int32).reshape(B, NSL, S_SL)
    g = [_sc_gather(W_tok, tok3[:, sl, :].reshape(-1)) for sl in range(NSL)]
    out = _combine_slice[0](lang_r, ltab, g[0], pe16)
    for sl in range(1, NSL):
        out = _combine_slice[sl](out, lang_r, ltab, g[sl], pe16)
    return out.reshape(B, S, D)


# speedup vs baseline: 3.7254x; 1.5656x over previous
import functools
import math

import jax
import jax.numpy as jnp
import ml_dtypes
import numpy as np
from jax import lax
from jax.experimental import pallas as pl
from jax.experimental.pallas import tpu as pltpu
from jax.experimental.pallas import tpu_sc as plsc

VOCAB = 100000
D = 768
NUM_LANG = 4
B = 4
S = 4096
N = B * S
SCALE = math.sqrt(D)

NSL = 2
S_SL = S // NSL
N_SL = B * S_SL

NC = 2
NS = 16
NW = NC * NS
R_PER_W = N_SL // NW
CHUNK = 32
NBUF = 4
PF = 2
NCHUNK = R_PER_W // CHUNK

BLK = 512
NBLK_SL = S_SL // BLK
NBLK = S // BLK


def _pos_table():
    pos = np.arange(0, S, dtype=np.float32)[:, None]
    div = np.exp(np.arange(0, D, 2, dtype=np.float32) * (-math.log(10000.0) / D))
    pe = np.zeros((S, D), dtype=np.float32)
    pe[:, 0::2] = np.sin(pos * div)
    pe[:, 1::2] = np.cos(pos * div)
    return pe.astype(ml_dtypes.bfloat16)


_PE16 = _pos_table()


def _proj_body(wl_ref, wp_ref, o_ref):
    o_ref[...] = lax.dot_general(
        wl_ref[...], wp_ref[...], (((1,), (1,)), ((), ())),
        preferred_element_type=jnp.float32)


_lang_proj = pl.pallas_call(
    _proj_body,
    out_shape=jax.ShapeDtypeStruct((NUM_LANG, D), jnp.float32),
)


@functools.partial(
    pl.kernel,
    out_type=jax.ShapeDtypeStruct((N_SL, D), jnp.float32),
    mesh=plsc.VectorSubcoreMesh(core_axis_name="c", subcore_axis_name="s"),
    scratch_types=(
        [pltpu.VMEM((R_PER_W,), jnp.int32)]
        + [pltpu.VMEM((CHUNK, D), jnp.float32)] * NBUF
        + [pltpu.SemaphoreType.DMA] * (2 * NBUF)
    ),
)
def _sc_gather(wtok_hbm, tokid_hbm, g_hbm, idx_v, *bufs_and_sems):
    bufs = bufs_and_sems[:NBUF]
    gsems = bufs_and_sems[NBUF:2 * NBUF]
    ssems = bufs_and_sems[2 * NBUF:]
    wid = lax.axis_index("s") * NC + lax.axis_index("c")
    base = wid * R_PER_W
    pltpu.sync_copy(tokid_hbm.at[pl.ds(base, R_PER_W)], idx_v)
    g_cp = [None] * NCHUNK
    st_cp = [None] * NCHUNK
    for c in range(PF):
        g_cp[c] = pltpu.async_copy(
            wtok_hbm.at[idx_v.at[pl.ds(c * CHUNK, CHUNK)]], bufs[c % NBUF],
            gsems[c % NBUF])
    for c in range(NCHUNK):
        k = c % NBUF
        if c + PF < NCHUNK:
            if c >= NBUF - PF:
                st_cp[c - (NBUF - PF)].wait()
            kk = (c + PF) % NBUF
            g_cp[c + PF] = pltpu.async_copy(
                wtok_hbm.at[idx_v.at[pl.ds((c + PF) * CHUNK, CHUNK)]],
                bufs[kk], gsems[kk])
        g_cp[c].wait()
        st_cp[c] = pltpu.async_copy(
            bufs[k], g_hbm.at[pl.ds(base + c * CHUNK, CHUNK)], ssems[k])
    for c in range(max(0, NCHUNK - NBUF), NCHUNK):
        st_cp[c].wait()


def _combine_body(lid_ref, ltab_ref, g_ref, pe_ref, o_ref):
    ids_row = lid_ref[0]
    oh = (lax.broadcasted_iota(jnp.int32, (NUM_LANG, BLK), 0)
          == jnp.broadcast_to(ids_row, (NUM_LANG, BLK))).astype(jnp.float32)
    lang = lax.dot_general(oh, ltab_ref[...], (((0,), (0,)), ((), ())),
                           preferred_element_type=jnp.float32)
    o_ref[...] = (g_ref[...] * SCALE + lang
                  + pe_ref[...].astype(jnp.float32))


def _make_combine(sl, aliased):
    kw = {}
    specs = [
        pl.BlockSpec((1, 1, BLK),
                     lambda i, b: (b * NBLK + sl * NBLK_SL + i, 0, 0)),
        pl.BlockSpec((NUM_LANG, D), lambda i, b: (0, 0)),
        pl.BlockSpec((BLK, D), lambda i, b: (b * NBLK_SL + i, 0)),
        pl.BlockSpec((BLK, D), lambda i, b: (sl * NBLK_SL + i, 0)),
    ]
    out_spec = pl.BlockSpec((BLK, D),
                            lambda i, b: (b * NBLK + sl * NBLK_SL + i, 0))
    if aliased:
        specs = [pl.BlockSpec(
            (BLK, D), lambda i, b: (b * NBLK + sl * NBLK_SL + i, 0))] + specs
        kw["input_output_aliases"] = {0: 0}

        def body(prev_ref, lid_ref, ltab_ref, g_ref, pe_ref, o_ref):
            del prev_ref
            _combine_body(lid_ref, ltab_ref, g_ref, pe_ref, o_ref)
    else:
        body = _combine_body

    return pl.pallas_call(
        body,
        grid=(NBLK_SL, B),
        in_specs=specs,
        out_specs=out_spec,
        out_shape=jax.ShapeDtypeStruct((N, D), jnp.float32),
        **kw,
    )


_combine_slice = [_make_combine(sl, aliased=(sl > 0)) for sl in range(NSL)]


def kernel(token_ids, lang_ids, W_tok, W_lang, W_proj):
    lang_r = lang_ids.reshape(-1).astype(jnp.int32).reshape(B * NBLK, 1, BLK)
    ltab = _lang_proj(W_lang, W_proj)
    pe16 = jnp.asarray(_PE16)
    tok3 = token_ids.astype(jnp.int32).reshape(B, NSL, S_SL)
    g = [_sc_gather(W_tok, tok3[:, sl, :].reshape(-1)) for sl in range(NSL)]
    out = _combine_slice[0](lang_r, ltab, g[0], pe16)
    for sl in range(1, NSL):
        out = _combine_slice[sl](out, lang_r, ltab, g[sl], pe16)
    return out.reshape(B, S, D)
